# Initial kernel scaffold; baseline (speedup 1.0000x reference)
import jax
import jax.numpy as jnp
from jax.experimental import pallas as pl


def kernel(mirna_enc, target_enc, node_type_emb, edge_type_emb, W_in, b_in, g_in, beta_in, Wl0, bl0, Wr0, br0, We0, att0, bias0, g0, beta0, Wl1, bl1, Wr1, br1, We1, att1, bias1, g1, beta1, W_out, b_out, g_out, beta_out):
    raise NotImplementedError("write your pallas kernel here")


# speedup vs baseline: 7.6844x; 7.6844x over previous
import functools

import jax
import jax.numpy as jnp
import numpy as np
from jax.experimental import pallas as pl

Lm, Lt = 30, 50
N = Lm + Lt
NODE_DIM = 256
HID = 128
HEADS = 4
HC = HID * HEADS
OUT_DIM = 128
B = 256
BS = 16
OFFSETS = (-2, -1, 0, 1, 2)
NEG = -1e30


def _layer_norm(x, g, b):
    mu = jnp.mean(x, axis=-1, keepdims=True)
    var = jnp.mean((x - mu) ** 2, axis=-1, keepdims=True)
    return (x - mu) * jax.lax.rsqrt(var + 1e-5) * g + b


def _gelu(x):
    return 0.5 * x * (1.0 + jax.lax.erf(x * np.float32(1.0 / np.sqrt(2.0))))


def _gat_stencil(x, row_mod, Wlr_ref, blr_ref, We_ref, A_ref, S_ref, bias_ref,
                 attrs):
    xlr = jnp.dot(x, Wlr_ref[...], preferred_element_type=jnp.float32) + blr_ref[...]
    xl = xlr[:, :HC]
    xr = xlr[:, HC:]
    ep3 = jnp.dot(attrs, We_ref[...], preferred_element_type=jnp.float32)
    A = A_ref[...]
    S = S_ref[...]

    chain = row_mod >= Lm
    shifts = []
    logits = []
    for o in OFFSETS:
        xs = xl if o == 0 else jnp.roll(xl, -o, axis=0)
        erow = 2 if o == 0 else (abs(o) - 1)
        msg = xs + xr + ep3[erow:erow + 1, :]
        msg = jnp.where(msg >= 0, msg, 0.2 * msg)
        lg = jnp.dot(msg, A, preferred_element_type=jnp.float32)
        ns = row_mod + o
        valid = (ns >= 0) & (ns < N) & ((ns >= Lm) == chain)
        lg = jnp.where(valid, lg, NEG)
        shifts.append(xs)
        logits.append(lg)
    amax = functools.reduce(jnp.maximum, logits)
    exs = [jnp.exp(l - amax) for l in logits]
    den = functools.reduce(jnp.add, exs) + 1e-16
    out = bias_ref[...]
    for ex, xs in zip(exs, shifts):
        aw = jnp.dot(ex / den, S, preferred_element_type=jnp.float32)
        out = out + aw * xs
    return out


def _model_kernel(x_ref, nte_ref, ete_ref,
                  W_in_ref, b_in_ref, g_in_ref, beta_in_ref,
                  Wlr0_ref, blr0_ref, We0_ref, A0_ref, S0_ref, bias0_ref, g0_ref, beta0_ref,
                  Wlr1_ref, blr1_ref, We1_ref, A1_ref, S1_ref, bias1_ref, g1_ref, beta1_ref,
                  W_out_ref, b_out_ref, g_out_ref, beta_out_ref,
                  o_ref):
    R = BS * N
    x = x_ref[...].reshape(R, NODE_DIM)
    row_mod = jax.lax.broadcasted_iota(jnp.int32, (R, 1), 0) % N
    emb0 = nte_ref[0:1, :]
    emb1 = nte_ref[1:2, :]
    x = x + jnp.where(row_mod < Lm, emb0, emb1)

    h = jnp.dot(x, W_in_ref[...], preferred_element_type=jnp.float32) + b_in_ref[...]
    h = _gelu(_layer_norm(h, g_in_ref[...], beta_in_ref[...]))

    e0 = ete_ref[0:1, :]
    e2 = ete_ref[2:3, :]
    eloop = (156.0 * e0 + 152.0 * e2) * np.float32(1.0 / 308.0)
    attrs = jnp.concatenate([e0, e2, eloop], axis=0)

    res = h
    h = _gat_stencil(h, row_mod, Wlr0_ref, blr0_ref, We0_ref, A0_ref, S0_ref,
                     bias0_ref, attrs)
    h = _gelu(_layer_norm(h + res, g0_ref[...], beta0_ref[...]))
    res = h
    h = _gat_stencil(h, row_mod, Wlr1_ref, blr1_ref, We1_ref, A1_ref, S1_ref,
                     bias1_ref, attrs)
    h = _layer_norm(h + res, g1_ref[...], beta1_ref[...])

    h3 = h.reshape(BS, N, HC)
    mean_pool = jnp.mean(h3, axis=1)
    max_pool = jnp.max(h3, axis=1)
    pooled = jnp.concatenate([mean_pool, max_pool], axis=-1)
    out = jnp.dot(pooled, W_out_ref[...], preferred_element_type=jnp.float32) + b_out_ref[...]
    o_ref[...] = _gelu(_layer_norm(out, g_out_ref[...], beta_out_ref[...]))


def kernel(mirna_enc, target_enc, node_type_emb, edge_type_emb, W_in, b_in,
           g_in, beta_in, Wl0, bl0, Wr0, br0, We0, att0, bias0, g0, beta0,
           Wl1, bl1, Wr1, br1, We1, att1, bias1, g1, beta1, W_out, b_out,
           g_out, beta_out):
    x_in = jnp.concatenate([mirna_enc, target_enc], axis=1)
    bsz = x_in.shape[0]

    Wlr0 = jnp.concatenate([Wl0, Wr0], axis=1)
    Wlr1 = jnp.concatenate([Wl1, Wr1], axis=1)
    blr0 = jnp.concatenate([bl0, br0])[None, :]
    blr1 = jnp.concatenate([bl1, br1])[None, :]
    lane = np.arange(HC)[:, None]
    head = np.arange(HEADS)[None, :]
    blk = jnp.asarray(lane // HID == head, jnp.float32)
    A0 = att0.reshape(HC)[:, None] * blk
    A1 = att1.reshape(HC)[:, None] * blk
    S = blk.T

    def row2(v):
        return v[None, :]

    full = lambda a: pl.BlockSpec(a.shape, lambda i: (0,) * a.ndim)
    operands = [
        (x_in, pl.BlockSpec((BS, N, NODE_DIM), lambda i: (i, 0, 0))),
        (node_type_emb, full(node_type_emb)),
        (edge_type_emb, full(edge_type_emb)),
        (W_in, full(W_in)),
        (row2(b_in), None), (row2(g_in), None), (row2(beta_in), None),
        (Wlr0, full(Wlr0)),
        (blr0, None), (We0, full(We0)), (A0, full(A0)), (S, full(S)),
        (row2(bias0), None), (row2(g0), None), (row2(beta0), None),
        (Wlr1, full(Wlr1)),
        (blr1, None), (We1, full(We1)), (A1, full(A1)), (S, full(S)),
        (row2(bias1), None), (row2(g1), None), (row2(beta1), None),
        (W_out, full(W_out)),
        (row2(b_out), None), (row2(g_out), None), (row2(beta_out), None),
    ]
    args = [a for a, _ in operands]
    specs = [s if s is not None else full(a) for a, s in operands]

    out = pl.pallas_call(
        _model_kernel,
        grid=(bsz // BS,),
        in_specs=specs,
        out_specs=pl.BlockSpec((BS, OUT_DIM), lambda i: (i, 0)),
        out_shape=jax.ShapeDtypeStruct((bsz, OUT_DIM), jnp.float32),
    )(*args)
    return out
